# baseline (speedup 1.0000x reference)
import jax
import jax.numpy as jnp
from jax.experimental import pallas as pl
from jax.experimental.pallas import tpu as pltpu

_THRESHOLD = 0.5
_ROWS = 8
_WIN = 24


def _fused_kernel(x_ref, psm_ref, gk_ref, out_ref, rate_ref, acc_ref):
    i = pl.program_id(0)
    n_i = pl.num_programs(0)
    N, C, R, W = x_ref.shape
    L, A, H, _ = psm_ref.shape

    start = 8 * jnp.minimum(jnp.maximum(i - 1, 0), (H - _WIN) // 8)
    off = 8 * i - start
    p = psm_ref[:, :, pl.ds(start, _WIN), :]
    m = jnp.max(jax.nn.sigmoid(p), axis=1)
    mp = jnp.pad(m, ((0, 0), (2, 2), (2, 2)))
    sm = jnp.zeros((L, _WIN, W), dtype=jnp.float32)
    for di in range(5):
        for dj in range(5):
            sm = sm + gk_ref[di, dj] * jax.lax.slice(
                mp, (0, di, dj), (L, di + _WIN, dj + W))
    mskw = jnp.where(sm > _THRESHOLD, 1.0, 0.0).astype(jnp.float32)
    msk = jnp.where(
        off == 0, jax.lax.slice(mskw, (0, 0, 0), (L, R, W)),
        jnp.where(
            off == 8, jax.lax.slice(mskw, (0, 8, 0), (L, 8 + R, W)),
            jax.lax.slice(mskw, (0, 16, 0), (L, 16 + R, W))))

    @pl.when(i == 0)
    def _():
        acc_ref[0] = 0.0

    acc_ref[0] += jnp.sum(msk)

    @pl.when(i == n_i - 1)
    def _():
        rate_ref[...] = (acc_ref[0] / (L * H * W)).reshape(1, 1)

    inv_sqrt_c = 1.0 / jnp.sqrt(jnp.float32(C))
    x0 = x_ref[0]
    s = [jnp.sum(x0 * x0, axis=0) * inv_sqrt_c]
    for n in range(1, N):
        dot = jnp.sum(x0 * x_ref[n], axis=0) * inv_sqrt_c
        s.append(msk[n] * dot)
    smax = s[0]
    for n in range(1, N):
        smax = jnp.maximum(smax, s[n])
    e = [jnp.exp(v - smax) for v in s]
    denom = e[0]
    for n in range(1, N):
        denom = denom + e[n]
    inv = 1.0 / denom
    out = (e[0] * inv)[None] * x0
    for n in range(1, N):
        out = out + (e[n] * inv * msk[n])[None] * x_ref[n]
    out_ref[...] = out


def kernel(x, psm_single, record_len, pairwise_t_matrix, trajectory, gauss_kernel):
    N, C, H, W = x.shape
    L = psm_single.shape[0]

    x_fuse, rate = pl.pallas_call(
        _fused_kernel,
        grid=(H // _ROWS,),
        in_specs=[
            pl.BlockSpec((N, C, _ROWS, W), lambda i: (0, 0, i, 0)),
            pl.BlockSpec(psm_single.shape, lambda i: (0, 0, 0, 0)),
            pl.BlockSpec(gauss_kernel.shape, lambda i: (0, 0)),
        ],
        out_specs=(
            pl.BlockSpec((C, _ROWS, W), lambda i: (0, i, 0)),
            pl.BlockSpec((1, 1), lambda i: (0, 0)),
        ),
        out_shape=(
            jax.ShapeDtypeStruct((C, H, W), jnp.float32),
            jax.ShapeDtypeStruct((1, 1), jnp.float32),
        ),
        scratch_shapes=[pltpu.SMEM((1,), jnp.float32)],
    )(x, psm_single, gauss_kernel)

    B = pairwise_t_matrix.shape[0]
    comm_rates = rate.reshape(()) / B
    return x_fuse[None], comm_rates


# speedup vs baseline: 1.2495x; 1.2495x over previous
import jax
import jax.numpy as jnp
from jax.experimental import pallas as pl
from jax.experimental.pallas import tpu as pltpu

_THRESHOLD = 0.5
_ROWS = 8


def _fused_kernel(x_ref, psm_ref, gk_ref, out_ref, rate_ref, mask_ref):
    i = pl.program_id(0)
    N, C, R, W = x_ref.shape
    L, _, H, _ = psm_ref.shape

    @pl.when(i == 0)
    def _():
        p = psm_ref[...]
        m = jnp.max(jax.nn.sigmoid(p), axis=1)
        mv = jnp.pad(m, ((0, 0), (2, 2), (0, 0)))
        vm = jnp.zeros((L, H, W), dtype=jnp.float32)
        for a in range(5):
            vm = vm + gk_ref[a, 2] * jax.lax.slice(mv, (0, a, 0), (L, a + H, W))
        vh = jnp.pad(vm, ((0, 0), (0, 0), (2, 2)))
        sm = jnp.zeros((L, H, W), dtype=jnp.float32)
        inv_center = 1.0 / gk_ref[2, 2]
        for b in range(5):
            sm = sm + (gk_ref[2, b] * inv_center) * jax.lax.slice(
                vh, (0, 0, b), (L, H, b + W))
        msk = jnp.where(sm > _THRESHOLD, 1.0, 0.0).astype(jnp.float32)
        rate_ref[...] = (jnp.sum(msk) / (L * H * W)).reshape(1, 1)
        mask_ref[...] = msk

    msk = mask_ref[:, pl.ds(i * R, R), :]

    inv_sqrt_c = 1.0 / jnp.sqrt(jnp.float32(C))
    x0 = x_ref[0]
    s = [jnp.sum(x0 * x0, axis=0) * inv_sqrt_c]
    for n in range(1, N):
        dot = jnp.sum(x0 * x_ref[n], axis=0) * inv_sqrt_c
        s.append(msk[n] * dot)
    smax = s[0]
    for n in range(1, N):
        smax = jnp.maximum(smax, s[n])
    e = [jnp.exp(v - smax) for v in s]
    denom = e[0]
    for n in range(1, N):
        denom = denom + e[n]
    inv = 1.0 / denom
    out = (e[0] * inv)[None] * x0
    for n in range(1, N):
        out = out + (e[n] * inv * msk[n])[None] * x_ref[n]
    out_ref[...] = out


def kernel(x, psm_single, record_len, pairwise_t_matrix, trajectory, gauss_kernel):
    N, C, H, W = x.shape
    L = psm_single.shape[0]

    x_fuse, rate = pl.pallas_call(
        _fused_kernel,
        grid=(H // _ROWS,),
        in_specs=[
            pl.BlockSpec((N, C, _ROWS, W), lambda i: (0, 0, i, 0)),
            pl.BlockSpec(psm_single.shape, lambda i: (0, 0, 0, 0)),
            pl.BlockSpec(gauss_kernel.shape, lambda i: (0, 0)),
        ],
        out_specs=(
            pl.BlockSpec((C, _ROWS, W), lambda i: (0, i, 0)),
            pl.BlockSpec((1, 1), lambda i: (0, 0)),
        ),
        out_shape=(
            jax.ShapeDtypeStruct((C, H, W), jnp.float32),
            jax.ShapeDtypeStruct((1, 1), jnp.float32),
        ),
        scratch_shapes=[pltpu.VMEM((L, H, W), jnp.float32)],
    )(x, psm_single, gauss_kernel)

    B = pairwise_t_matrix.shape[0]
    comm_rates = rate.reshape(()) / B
    return x_fuse[None], comm_rates
